# baseline (speedup 1.0000x reference)
import functools

import jax
import jax.numpy as jnp
from jax import lax
from jax.experimental import pallas as pl
from jax.experimental.pallas import tpu as pltpu
from jax.experimental.pallas import tpu_sc as plsc

N_NODES = 10000
D = 128
N_EDGES = 320000
NT = 16
B = 100
EROWS = N_EDGES // B
RPT = EROWS // NT
PAIRS = RPT // 2
CH = 20
NCH = RPT // CH
ROWS_PT = 624
TAIL_BASE = ROWS_PT * NT
TAIL = N_NODES - TAIL_BASE
DEGW = 16

_deg_struct = jax.ShapeDtypeStruct((N_NODES, DEGW), jnp.float32)
_agg_struct = jax.ShapeDtypeStruct((N_NODES, D), jnp.float32)


@functools.cache
def _sc_kernels():
    mesh = plsc.VectorSubcoreMesh(core_axis_name="c", subcore_axis_name="s")
    params = pltpu.CompilerParams(use_tc_tiling_on_sc=False)

    @functools.partial(
        pl.kernel,
        out_type=(_deg_struct, _deg_struct, _deg_struct, _deg_struct),
        mesh=mesh,
        compiler_params=params,
        scratch_types=(
            pltpu.VMEM_SHARED((N_NODES, DEGW), jnp.float32),
            pltpu.VMEM_SHARED((N_NODES, DEGW), jnp.float32),
            pltpu.VMEM((RPT, B), jnp.int32),
            pltpu.VMEM((RPT, B), jnp.int32),
            pltpu.VMEM((B, DEGW), jnp.float32),
            pltpu.SemaphoreType.DMA,
        ),
    )
    def sc_degrees(src0, dst0, src1, dst1, ones_hbm, zeros_hbm,
                   outdeg0, indeg0, outdeg1, indeg1,
                   deg_out_sh, deg_in_sh, src_v, dst_v, ones_v, dsem):
        c = lax.axis_index("c")
        s = lax.axis_index("s")
        sl = pl.ds(s * ROWS_PT, ROWS_PT)
        tsl = pl.ds(TAIL_BASE, TAIL)
        pltpu.sync_copy(zeros_hbm.at[pl.ds(0, ROWS_PT)], deg_out_sh.at[sl])
        pltpu.sync_copy(zeros_hbm.at[pl.ds(0, ROWS_PT)], deg_in_sh.at[sl])

        @pl.when(s == NT - 1)
        def _():
            pltpu.sync_copy(zeros_hbm.at[pl.ds(0, TAIL)], deg_out_sh.at[tsl])
            pltpu.sync_copy(zeros_hbm.at[pl.ds(0, TAIL)], deg_in_sh.at[tsl])

        pltpu.sync_copy(ones_hbm, ones_v)
        esl = pl.ds(s * RPT, RPT)

        @pl.when(c == 0)
        def _():
            pltpu.sync_copy(src0.at[esl], src_v)
            pltpu.sync_copy(dst0.at[esl], dst_v)

        @pl.when(c == 1)
        def _():
            pltpu.sync_copy(src1.at[esl], src_v)
            pltpu.sync_copy(dst1.at[esl], dst_v)

        plsc.subcore_barrier()

        def fire(q):
            for t in range(4):
                pltpu.async_copy(ones_v, deg_out_sh.at[src_v.at[4 * q + t]],
                                 dsem, add=True)
                pltpu.async_copy(ones_v, deg_in_sh.at[dst_v.at[4 * q + t]],
                                 dsem, add=True)

        def drain():
            for _t in range(8):
                pltpu.make_async_copy(ones_v, deg_out_sh.at[src_v.at[0]],
                                      dsem).wait()

        fire(0)

        @pl.loop(1, RPT // 4)
        def _(q):
            fire(q)
            drain()

        drain()

        plsc.subcore_barrier()

        @pl.when(c == 0)
        def _():
            pltpu.sync_copy(deg_out_sh.at[sl], outdeg0.at[sl])
            pltpu.sync_copy(deg_in_sh.at[sl], indeg0.at[sl])

            @pl.when(s == NT - 1)
            def _():
                pltpu.sync_copy(deg_out_sh.at[tsl], outdeg0.at[tsl])
                pltpu.sync_copy(deg_in_sh.at[tsl], indeg0.at[tsl])

        @pl.when(c == 1)
        def _():
            pltpu.sync_copy(deg_out_sh.at[sl], outdeg1.at[sl])
            pltpu.sync_copy(deg_in_sh.at[sl], indeg1.at[sl])

            @pl.when(s == NT - 1)
            def _():
                pltpu.sync_copy(deg_out_sh.at[tsl], outdeg1.at[tsl])
                pltpu.sync_copy(deg_in_sh.at[tsl], indeg1.at[tsl])

    @functools.partial(
        pl.kernel,
        out_type=(_agg_struct, _agg_struct),
        mesh=mesh,
        compiler_params=params,
        scratch_types=(
            pltpu.VMEM_SHARED((N_NODES, D), jnp.float32),
            pltpu.VMEM((CH, B), jnp.int32),
            pltpu.VMEM((CH, B), jnp.int32),
            pltpu.VMEM((B, D), jnp.float32),
            pltpu.VMEM((B, D), jnp.float32),
            pltpu.SemaphoreType.DMA,
            pltpu.SemaphoreType.DMA,
            pltpu.SemaphoreType.DMA,
            pltpu.SemaphoreType.DMA,
        ),
    )
    def sc_scatter(y0, y1, src0, dst0, src1, dst1, zeros_hbm,
                   agg0, agg1,
                   agg_sh, src_v, dst_v, rows0, rows1, gs0, gs1, ss0, ss1):
        c = lax.axis_index("c")
        s = lax.axis_index("s")
        sl = pl.ds(s * ROWS_PT, ROWS_PT)
        tsl = pl.ds(TAIL_BASE, TAIL)
        pltpu.sync_copy(zeros_hbm.at[pl.ds(0, ROWS_PT)], agg_sh.at[sl])

        @pl.when(s == NT - 1)
        def _():
            pltpu.sync_copy(zeros_hbm.at[pl.ds(0, TAIL)], agg_sh.at[tsl])

        plsc.subcore_barrier()

        erow0 = s * RPT

        def edge_loop(y_hbm, src_hbm, dst_hbm):
            def gather(j, buf, sem):
                pltpu.async_copy(y_hbm.at[src_v.at[j]], buf, sem)

            def wait_g(buf, sem):
                pltpu.make_async_copy(y_hbm.at[src_v.at[0]], buf, sem).wait()

            def scat(j, buf, sem):
                pltpu.async_copy(buf, agg_sh.at[dst_v.at[j]], sem, add=True)

            def wait_s(buf, sem):
                pltpu.make_async_copy(buf, agg_sh.at[dst_v.at[0]], sem).wait()

            @pl.loop(0, NCH)
            def _(k):
                csl = pl.ds(erow0 + k * CH, CH)
                pltpu.sync_copy(src_hbm.at[csl], src_v)
                pltpu.sync_copy(dst_hbm.at[csl], dst_v)

                gather(0, rows0, gs0)
                gather(1, rows1, gs1)

                @pl.loop(0, CH // 2)
                def _(i):
                    j0 = 2 * i
                    wait_g(rows0, gs0)
                    scat(j0, rows0, ss0)
                    wait_g(rows1, gs1)
                    scat(j0 + 1, rows1, ss1)

                    @pl.when(i < CH // 2 - 1)
                    def _():
                        wait_s(rows0, ss0)
                        gather(j0 + 2, rows0, gs0)
                        wait_s(rows1, ss1)
                        gather(j0 + 3, rows1, gs1)

                wait_s(rows0, ss0)
                wait_s(rows1, ss1)

        @pl.when(c == 0)
        def _():
            edge_loop(y0, src0, dst0)

        @pl.when(c == 1)
        def _():
            edge_loop(y1, src1, dst1)

        plsc.subcore_barrier()

        @pl.when(c == 0)
        def _():
            pltpu.sync_copy(agg_sh.at[sl], agg0.at[sl])

            @pl.when(s == NT - 1)
            def _():
                pltpu.sync_copy(agg_sh.at[tsl], agg0.at[tsl])

        @pl.when(c == 1)
        def _():
            pltpu.sync_copy(agg_sh.at[sl], agg1.at[sl])

            @pl.when(s == NT - 1)
            def _():
                pltpu.sync_copy(agg_sh.at[tsl], agg1.at[tsl])

    return sc_degrees, sc_scatter


_RB = 1000


def _y_body(x_ref, d0_ref, d1_ref, w0_ref, w1_ref, y0_ref, y1_ref):
    c0 = lax.rsqrt(jnp.maximum(d0_ref[:, 0:1], 1.0))
    c1 = lax.rsqrt(jnp.maximum(d1_ref[:, 0:1], 1.0))
    xb = x_ref[...]
    y0_ref[...] = jnp.dot(xb * c0, w0_ref[...], preferred_element_type=jnp.float32)
    y1_ref[...] = jnp.dot(xb * c1, w1_ref[...], preferred_element_type=jnp.float32)


def _tc_prepare_y(x, d0, d1, W0, W1):
    return pl.pallas_call(
        _y_body,
        grid=(N_NODES // _RB,),
        in_specs=[
            pl.BlockSpec((_RB, D), lambda i: (i, 0)),
            pl.BlockSpec((_RB, DEGW), lambda i: (i, 0)),
            pl.BlockSpec((_RB, DEGW), lambda i: (i, 0)),
            pl.BlockSpec((D, D), lambda i: (0, 0)),
            pl.BlockSpec((D, D), lambda i: (0, 0)),
        ],
        out_specs=[
            pl.BlockSpec((_RB, D), lambda i: (i, 0)),
            pl.BlockSpec((_RB, D), lambda i: (i, 0)),
        ],
        out_shape=[
            jax.ShapeDtypeStruct((N_NODES, D), jnp.float32),
            jax.ShapeDtypeStruct((N_NODES, D), jnp.float32),
        ],
    )(x, d0, d1, W0, W1)


def _fin_body(a0_ref, a1_ref, d0_ref, d1_ref, b0_ref, b1_ref, o_ref):
    s0 = lax.rsqrt(jnp.maximum(d0_ref[:, 0:1], 1.0))
    s1 = lax.rsqrt(jnp.maximum(d1_ref[:, 0:1], 1.0))
    o_ref[...] = a0_ref[...] * s0 + a1_ref[...] * s1 + b0_ref[...] + b1_ref[...]


def _tc_finalize(agg0, agg1, d0, d1, b0, b1):
    return pl.pallas_call(
        _fin_body,
        grid=(N_NODES // _RB,),
        in_specs=[
            pl.BlockSpec((_RB, D), lambda i: (i, 0)),
            pl.BlockSpec((_RB, D), lambda i: (i, 0)),
            pl.BlockSpec((_RB, DEGW), lambda i: (i, 0)),
            pl.BlockSpec((_RB, DEGW), lambda i: (i, 0)),
            pl.BlockSpec((1, D), lambda i: (0, 0)),
            pl.BlockSpec((1, D), lambda i: (0, 0)),
        ],
        out_specs=pl.BlockSpec((_RB, D), lambda i: (i, 0)),
        out_shape=jax.ShapeDtypeStruct((N_NODES, D), jnp.float32),
    )(agg0, agg1, d0, d1, b0, b1)


def kernel(x, edge_index_rel0, edge_index_rel1, W0, b0, W1, b1):
    src0 = edge_index_rel0[0].astype(jnp.int32).reshape(EROWS, B)
    dst0 = edge_index_rel0[1].astype(jnp.int32).reshape(EROWS, B)
    src1 = edge_index_rel1[0].astype(jnp.int32).reshape(EROWS, B)
    dst1 = edge_index_rel1[1].astype(jnp.int32).reshape(EROWS, B)
    ones_hbm = jnp.ones((B, DEGW), jnp.float32)
    zeros_deg = jnp.zeros((ROWS_PT, DEGW), jnp.float32)
    zeros_agg = jnp.zeros((ROWS_PT, D), jnp.float32)

    sc_degrees, sc_scatter = _sc_kernels()
    outdeg0, indeg0, outdeg1, indeg1 = sc_degrees(
        src0, dst0, src1, dst1, ones_hbm, zeros_deg)
    y0, y1 = _tc_prepare_y(x, outdeg0, outdeg1, W0, W1)
    agg0, agg1 = sc_scatter(y0, y1, src0, dst0, src1, dst1, zeros_agg)
    return _tc_finalize(agg0, agg1, indeg0, indeg1,
                        b0.reshape(1, D), b1.reshape(1, D))


# speedup vs baseline: 1.3148x; 1.3148x over previous
import functools

import jax
import jax.numpy as jnp
from jax import lax
from jax.experimental import pallas as pl
from jax.experimental.pallas import tpu as pltpu
from jax.experimental.pallas import tpu_sc as plsc

N_NODES = 10000
D = 128
N_EDGES = 320000
NT = 16
B = 100
EROWS = N_EDGES // B
RPT = EROWS // NT
BS = 40
EROWS_S = N_EDGES // BS
RPT_S = EROWS_S // NT
NB = 5
CH = 50
NCH = RPT_S // CH
ROUNDS = CH // NB
ROWS_PT = 624
TAIL_BASE = ROWS_PT * NT
TAIL = N_NODES - TAIL_BASE
DEGW = 16

_deg_struct = jax.ShapeDtypeStruct((N_NODES, DEGW), jnp.float32)
_agg_struct = jax.ShapeDtypeStruct((N_NODES, D), jnp.float32)


@functools.cache
def _sc_kernels():
    mesh = plsc.VectorSubcoreMesh(core_axis_name="c", subcore_axis_name="s")
    params = pltpu.CompilerParams(use_tc_tiling_on_sc=False)

    @functools.partial(
        pl.kernel,
        out_type=(_deg_struct, _deg_struct, _deg_struct, _deg_struct),
        mesh=mesh,
        compiler_params=params,
        scratch_types=(
            pltpu.VMEM_SHARED((N_NODES, DEGW), jnp.float32),
            pltpu.VMEM_SHARED((N_NODES, DEGW), jnp.float32),
            pltpu.VMEM((RPT, B), jnp.int32),
            pltpu.VMEM((RPT, B), jnp.int32),
            pltpu.VMEM((B, DEGW), jnp.float32),
            pltpu.SemaphoreType.DMA,
        ),
    )
    def sc_degrees(src0, dst0, src1, dst1, ones_hbm, zeros_hbm,
                   outdeg0, indeg0, outdeg1, indeg1,
                   deg_out_sh, deg_in_sh, src_v, dst_v, ones_v, dsem):
        c = lax.axis_index("c")
        s = lax.axis_index("s")
        sl = pl.ds(s * ROWS_PT, ROWS_PT)
        tsl = pl.ds(TAIL_BASE, TAIL)
        pltpu.sync_copy(zeros_hbm.at[pl.ds(0, ROWS_PT)], deg_out_sh.at[sl])
        pltpu.sync_copy(zeros_hbm.at[pl.ds(0, ROWS_PT)], deg_in_sh.at[sl])

        @pl.when(s == NT - 1)
        def _():
            pltpu.sync_copy(zeros_hbm.at[pl.ds(0, TAIL)], deg_out_sh.at[tsl])
            pltpu.sync_copy(zeros_hbm.at[pl.ds(0, TAIL)], deg_in_sh.at[tsl])

        pltpu.sync_copy(ones_hbm, ones_v)
        esl = pl.ds(s * RPT, RPT)

        @pl.when(c == 0)
        def _():
            pltpu.sync_copy(src0.at[esl], src_v)
            pltpu.sync_copy(dst0.at[esl], dst_v)

        @pl.when(c == 1)
        def _():
            pltpu.sync_copy(src1.at[esl], src_v)
            pltpu.sync_copy(dst1.at[esl], dst_v)

        plsc.subcore_barrier()

        def fire(q):
            for t in range(4):
                pltpu.async_copy(ones_v, deg_out_sh.at[src_v.at[4 * q + t]],
                                 dsem, add=True)
                pltpu.async_copy(ones_v, deg_in_sh.at[dst_v.at[4 * q + t]],
                                 dsem, add=True)

        def drain():
            for _t in range(8):
                pltpu.make_async_copy(ones_v, deg_out_sh.at[src_v.at[0]],
                                      dsem).wait()

        fire(0)
        fire(1)

        @pl.loop(2, RPT // 4)
        def _(q):
            fire(q)
            drain()

        drain()
        drain()

        plsc.subcore_barrier()

        @pl.when(c == 0)
        def _():
            pltpu.sync_copy(deg_out_sh.at[sl], outdeg0.at[sl])
            pltpu.sync_copy(deg_in_sh.at[sl], indeg0.at[sl])

            @pl.when(s == NT - 1)
            def _():
                pltpu.sync_copy(deg_out_sh.at[tsl], outdeg0.at[tsl])
                pltpu.sync_copy(deg_in_sh.at[tsl], indeg0.at[tsl])

        @pl.when(c == 1)
        def _():
            pltpu.sync_copy(deg_out_sh.at[sl], outdeg1.at[sl])
            pltpu.sync_copy(deg_in_sh.at[sl], indeg1.at[sl])

            @pl.when(s == NT - 1)
            def _():
                pltpu.sync_copy(deg_out_sh.at[tsl], outdeg1.at[tsl])
                pltpu.sync_copy(deg_in_sh.at[tsl], indeg1.at[tsl])

    scatter_scratch = (
        [pltpu.VMEM_SHARED((N_NODES, D), jnp.float32)]
        + [pltpu.VMEM((CH, BS), jnp.int32) for _ in range(4)]
        + [pltpu.VMEM((BS, D), jnp.float32) for _ in range(NB)]
        + [pltpu.SemaphoreType.DMA for _ in range(2 * NB + 2)]
    )

    @functools.partial(
        pl.kernel,
        out_type=(_agg_struct, _agg_struct),
        mesh=mesh,
        compiler_params=params,
        scratch_types=tuple(scatter_scratch),
    )
    def sc_scatter(y0, y1, src0, dst0, src1, dst1, zeros_hbm,
                   agg0, agg1,
                   agg_sh, srcA, dstA, srcB, dstB,
                   r0, r1, r2, r3, r4,
                   g0, g1, g2, g3, g4,
                   s0, s1, s2, s3, s4, isA, isB):
        c = lax.axis_index("c")
        s = lax.axis_index("s")
        bufs = (r0, r1, r2, r3, r4)
        gsems = (g0, g1, g2, g3, g4)
        ssems = (s0, s1, s2, s3, s4)
        sl = pl.ds(s * ROWS_PT, ROWS_PT)
        tsl = pl.ds(TAIL_BASE, TAIL)
        pltpu.sync_copy(zeros_hbm.at[pl.ds(0, ROWS_PT)], agg_sh.at[sl])

        @pl.when(s == NT - 1)
        def _():
            pltpu.sync_copy(zeros_hbm.at[pl.ds(0, TAIL)], agg_sh.at[tsl])

        plsc.subcore_barrier()

        erow0 = s * RPT_S

        def edge_loop(y_hbm, src_hbm, dst_hbm):
            def gather(sv, jj, buf, sem):
                pltpu.async_copy(y_hbm.at[sv.at[jj]], buf, sem)

            def wait_g(buf, sem):
                pltpu.make_async_copy(y_hbm.at[srcA.at[0]], buf, sem).wait()

            def scat(dv, jj, buf, sem):
                pltpu.async_copy(buf, agg_sh.at[dv.at[jj]], sem, add=True)

            def wait_s(buf, sem):
                pltpu.make_async_copy(buf, agg_sh.at[dstA.at[0]], sem).wait()

            def prefetch(k, sv, dv, isem):
                csl = pl.ds(erow0 + k * CH, CH)
                pltpu.async_copy(src_hbm.at[csl], sv, isem)
                pltpu.async_copy(dst_hbm.at[csl], dv, isem)

            def wait_idx(sv, dv, isem):
                pltpu.make_async_copy(src_hbm.at[pl.ds(0, CH)], sv, isem).wait()
                pltpu.make_async_copy(src_hbm.at[pl.ds(0, CH)], dv, isem).wait()

            def rounds(sv, dv):
                for b in range(NB):
                    gather(sv, b, bufs[b], gsems[b])

                @pl.loop(0, ROUNDS)
                def _(r):
                    for b in range(NB):
                        jj = NB * r + b
                        wait_g(bufs[b], gsems[b])
                        scat(dv, jj, bufs[b], ssems[b])

                        @pl.when(r < ROUNDS - 1)
                        def _(b=b, jj=jj):
                            wait_s(bufs[b], ssems[b])
                            gather(sv, jj + NB, bufs[b], gsems[b])

                for b in range(NB):
                    wait_s(bufs[b], ssems[b])

            csl0 = pl.ds(erow0, CH)
            pltpu.sync_copy(src_hbm.at[csl0], srcA)
            pltpu.sync_copy(dst_hbm.at[csl0], dstA)

            @pl.loop(0, NCH // 2)
            def _(m):
                k0 = 2 * m

                @pl.when(m > 0)
                def _():
                    wait_idx(srcA, dstA, isA)

                prefetch(k0 + 1, srcB, dstB, isB)
                rounds(srcA, dstA)

                wait_idx(srcB, dstB, isB)

                @pl.when(m < NCH // 2 - 1)
                def _():
                    prefetch(k0 + 2, srcA, dstA, isA)

                rounds(srcB, dstB)

        @pl.when(c == 0)
        def _():
            edge_loop(y0, src0, dst0)

        @pl.when(c == 1)
        def _():
            edge_loop(y1, src1, dst1)

        plsc.subcore_barrier()

        @pl.when(c == 0)
        def _():
            pltpu.sync_copy(agg_sh.at[sl], agg0.at[sl])

            @pl.when(s == NT - 1)
            def _():
                pltpu.sync_copy(agg_sh.at[tsl], agg0.at[tsl])

        @pl.when(c == 1)
        def _():
            pltpu.sync_copy(agg_sh.at[sl], agg1.at[sl])

            @pl.when(s == NT - 1)
            def _():
                pltpu.sync_copy(agg_sh.at[tsl], agg1.at[tsl])

    return sc_degrees, sc_scatter


_RB = 1000


def _y_body(x_ref, d0_ref, d1_ref, w0_ref, w1_ref, y0_ref, y1_ref):
    c0 = lax.rsqrt(jnp.maximum(d0_ref[:, 0:1], 1.0))
    c1 = lax.rsqrt(jnp.maximum(d1_ref[:, 0:1], 1.0))
    xb = x_ref[...]
    y0_ref[...] = jnp.dot(xb * c0, w0_ref[...], preferred_element_type=jnp.float32)
    y1_ref[...] = jnp.dot(xb * c1, w1_ref[...], preferred_element_type=jnp.float32)


def _tc_prepare_y(x, d0, d1, W0, W1):
    return pl.pallas_call(
        _y_body,
        grid=(N_NODES // _RB,),
        in_specs=[
            pl.BlockSpec((_RB, D), lambda i: (i, 0)),
            pl.BlockSpec((_RB, DEGW), lambda i: (i, 0)),
            pl.BlockSpec((_RB, DEGW), lambda i: (i, 0)),
            pl.BlockSpec((D, D), lambda i: (0, 0)),
            pl.BlockSpec((D, D), lambda i: (0, 0)),
        ],
        out_specs=[
            pl.BlockSpec((_RB, D), lambda i: (i, 0)),
            pl.BlockSpec((_RB, D), lambda i: (i, 0)),
        ],
        out_shape=[
            jax.ShapeDtypeStruct((N_NODES, D), jnp.float32),
            jax.ShapeDtypeStruct((N_NODES, D), jnp.float32),
        ],
    )(x, d0, d1, W0, W1)


def _fin_body(a0_ref, a1_ref, d0_ref, d1_ref, b0_ref, b1_ref, o_ref):
    s0 = lax.rsqrt(jnp.maximum(d0_ref[:, 0:1], 1.0))
    s1 = lax.rsqrt(jnp.maximum(d1_ref[:, 0:1], 1.0))
    o_ref[...] = a0_ref[...] * s0 + a1_ref[...] * s1 + b0_ref[...] + b1_ref[...]


def _tc_finalize(agg0, agg1, d0, d1, b0, b1):
    return pl.pallas_call(
        _fin_body,
        grid=(N_NODES // _RB,),
        in_specs=[
            pl.BlockSpec((_RB, D), lambda i: (i, 0)),
            pl.BlockSpec((_RB, D), lambda i: (i, 0)),
            pl.BlockSpec((_RB, DEGW), lambda i: (i, 0)),
            pl.BlockSpec((_RB, DEGW), lambda i: (i, 0)),
            pl.BlockSpec((1, D), lambda i: (0, 0)),
            pl.BlockSpec((1, D), lambda i: (0, 0)),
        ],
        out_specs=pl.BlockSpec((_RB, D), lambda i: (i, 0)),
        out_shape=jax.ShapeDtypeStruct((N_NODES, D), jnp.float32),
    )(agg0, agg1, d0, d1, b0, b1)


def kernel(x, edge_index_rel0, edge_index_rel1, W0, b0, W1, b1):
    e0 = edge_index_rel0.astype(jnp.int32)
    e1 = edge_index_rel1.astype(jnp.int32)
    src0 = e0[0].reshape(EROWS, B)
    dst0 = e0[1].reshape(EROWS, B)
    src1 = e1[0].reshape(EROWS, B)
    dst1 = e1[1].reshape(EROWS, B)
    src0s = e0[0].reshape(EROWS_S, BS)
    dst0s = e0[1].reshape(EROWS_S, BS)
    src1s = e1[0].reshape(EROWS_S, BS)
    dst1s = e1[1].reshape(EROWS_S, BS)
    ones_hbm = jnp.ones((B, DEGW), jnp.float32)
    zeros_deg = jnp.zeros((ROWS_PT, DEGW), jnp.float32)
    zeros_agg = jnp.zeros((ROWS_PT, D), jnp.float32)

    sc_degrees, sc_scatter = _sc_kernels()
    outdeg0, indeg0, outdeg1, indeg1 = sc_degrees(
        src0, dst0, src1, dst1, ones_hbm, zeros_deg)
    y0, y1 = _tc_prepare_y(x, outdeg0, outdeg1, W0, W1)
    agg0, agg1 = sc_scatter(y0, y1, src0s, dst0s, src1s, dst1s, zeros_agg)
    return _tc_finalize(agg0, agg1, indeg0, indeg1,
                        b0.reshape(1, D), b1.reshape(1, D))


# speedup vs baseline: 1.5173x; 1.1540x over previous
import functools

import jax
import jax.numpy as jnp
from jax import lax
from jax.experimental import pallas as pl
from jax.experimental.pallas import tpu as pltpu
from jax.experimental.pallas import tpu_sc as plsc

N_NODES = 10000
D = 128
N_EDGES = 320000
NT = 16
EPT = N_EDGES // NT
BD = 80
RPTD = EPT // BD
FIRE = 5
QD = RPTD // FIRE
BS = 40
RPT_S = EPT // BS
NB = 5
CH = 50
CHE = CH * BS
NCH = RPT_S // CH
ROUNDS = CH // NB
ROWS_PT = 624
TAIL_BASE = ROWS_PT * NT
TAIL = N_NODES - TAIL_BASE
DEGW = 16

_deg_struct = jax.ShapeDtypeStruct((N_NODES, DEGW), jnp.float32)
_agg_struct = jax.ShapeDtypeStruct((N_NODES, D), jnp.float32)


@functools.cache
def _sc_kernels():
    mesh = plsc.VectorSubcoreMesh(core_axis_name="c", subcore_axis_name="s")
    params = pltpu.CompilerParams(use_tc_tiling_on_sc=False)

    @functools.partial(
        pl.kernel,
        out_type=(_deg_struct, _deg_struct, _deg_struct, _deg_struct),
        mesh=mesh,
        compiler_params=params,
        scratch_types=(
            pltpu.VMEM_SHARED((N_NODES, DEGW), jnp.float32),
            pltpu.VMEM_SHARED((N_NODES, DEGW), jnp.float32),
            pltpu.VMEM((EPT,), jnp.int32),
            pltpu.VMEM((EPT,), jnp.int32),
            pltpu.VMEM((BD, DEGW), jnp.float32),
            pltpu.SemaphoreType.DMA,
        ),
    )
    def sc_degrees(e0, e1, ones_hbm, zeros_hbm,
                   outdeg0, indeg0, outdeg1, indeg1,
                   deg_out_sh, deg_in_sh, src_v, dst_v, ones_v, dsem):
        c = lax.axis_index("c")
        s = lax.axis_index("s")
        sl = pl.ds(s * ROWS_PT, ROWS_PT)
        tsl = pl.ds(TAIL_BASE, TAIL)
        pltpu.sync_copy(zeros_hbm.at[pl.ds(0, ROWS_PT)], deg_out_sh.at[sl])
        pltpu.sync_copy(zeros_hbm.at[pl.ds(0, ROWS_PT)], deg_in_sh.at[sl])

        @pl.when(s == NT - 1)
        def _():
            pltpu.sync_copy(zeros_hbm.at[pl.ds(0, TAIL)], deg_out_sh.at[tsl])
            pltpu.sync_copy(zeros_hbm.at[pl.ds(0, TAIL)], deg_in_sh.at[tsl])

        pltpu.sync_copy(ones_hbm, ones_v)
        esl = pl.ds(s * EPT, EPT)

        @pl.when(c == 0)
        def _():
            pltpu.sync_copy(e0.at[0, esl], src_v)
            pltpu.sync_copy(e0.at[1, esl], dst_v)

        @pl.when(c == 1)
        def _():
            pltpu.sync_copy(e1.at[0, esl], src_v)
            pltpu.sync_copy(e1.at[1, esl], dst_v)

        plsc.subcore_barrier()

        def fire(q):
            for t in range(FIRE):
                isl = pl.ds((FIRE * q + t) * BD, BD)
                pltpu.async_copy(ones_v, deg_out_sh.at[src_v.at[isl]],
                                 dsem, add=True)
                pltpu.async_copy(ones_v, deg_in_sh.at[dst_v.at[isl]],
                                 dsem, add=True)

        def drain():
            for _t in range(2 * FIRE):
                pltpu.make_async_copy(ones_v, deg_out_sh.at[src_v.at[pl.ds(0, BD)]],
                                      dsem).wait()

        fire(0)

        @pl.loop(1, QD)
        def _(q):
            fire(q)
            drain()

        drain()

        plsc.subcore_barrier()

        @pl.when(c == 0)
        def _():
            pltpu.sync_copy(deg_out_sh.at[sl], outdeg0.at[sl])
            pltpu.sync_copy(deg_in_sh.at[sl], indeg0.at[sl])

            @pl.when(s == NT - 1)
            def _():
                pltpu.sync_copy(deg_out_sh.at[tsl], outdeg0.at[tsl])
                pltpu.sync_copy(deg_in_sh.at[tsl], indeg0.at[tsl])

        @pl.when(c == 1)
        def _():
            pltpu.sync_copy(deg_out_sh.at[sl], outdeg1.at[sl])
            pltpu.sync_copy(deg_in_sh.at[sl], indeg1.at[sl])

            @pl.when(s == NT - 1)
            def _():
                pltpu.sync_copy(deg_out_sh.at[tsl], outdeg1.at[tsl])
                pltpu.sync_copy(deg_in_sh.at[tsl], indeg1.at[tsl])

    scatter_scratch = (
        [pltpu.VMEM_SHARED((N_NODES, D), jnp.float32)]
        + [pltpu.VMEM((CHE,), jnp.int32) for _ in range(4)]
        + [pltpu.VMEM((BS, D), jnp.float32) for _ in range(NB)]
        + [pltpu.SemaphoreType.DMA for _ in range(2 * NB + 2)]
    )

    @functools.partial(
        pl.kernel,
        out_type=(_agg_struct, _agg_struct),
        mesh=mesh,
        compiler_params=params,
        scratch_types=tuple(scatter_scratch),
    )
    def sc_scatter(y0, y1, e0, e1, zeros_hbm,
                   agg0, agg1,
                   agg_sh, srcA, dstA, srcB, dstB,
                   r0, r1, r2, r3, r4,
                   g0, g1, g2, g3, g4,
                   s0, s1, s2, s3, s4, isA, isB):
        c = lax.axis_index("c")
        s = lax.axis_index("s")
        bufs = (r0, r1, r2, r3, r4)
        gsems = (g0, g1, g2, g3, g4)
        ssems = (s0, s1, s2, s3, s4)
        sl = pl.ds(s * ROWS_PT, ROWS_PT)
        tsl = pl.ds(TAIL_BASE, TAIL)
        pltpu.sync_copy(zeros_hbm.at[pl.ds(0, ROWS_PT)], agg_sh.at[sl])

        @pl.when(s == NT - 1)
        def _():
            pltpu.sync_copy(zeros_hbm.at[pl.ds(0, TAIL)], agg_sh.at[tsl])

        plsc.subcore_barrier()

        ebase = s * EPT

        def edge_loop(y_hbm, e_hbm):
            def gather(sv, jj, buf, sem):
                pltpu.async_copy(y_hbm.at[sv.at[pl.ds(jj * BS, BS)]], buf, sem)

            def wait_g(buf, sem):
                pltpu.make_async_copy(y_hbm.at[srcA.at[pl.ds(0, BS)]],
                                      buf, sem).wait()

            def scat(dv, jj, buf, sem):
                pltpu.async_copy(buf, agg_sh.at[dv.at[pl.ds(jj * BS, BS)]],
                                 sem, add=True)

            def wait_s(buf, sem):
                pltpu.make_async_copy(buf, agg_sh.at[dstA.at[pl.ds(0, BS)]],
                                      sem).wait()

            def prefetch(k, sv, dv, isem):
                csl = pl.ds(ebase + k * CHE, CHE)
                pltpu.async_copy(e_hbm.at[0, csl], sv, isem)
                pltpu.async_copy(e_hbm.at[1, csl], dv, isem)

            def wait_idx(sv, dv, isem):
                pltpu.make_async_copy(e_hbm.at[0, pl.ds(0, CHE)], sv, isem).wait()
                pltpu.make_async_copy(e_hbm.at[0, pl.ds(0, CHE)], dv, isem).wait()

            def rounds(sv, dv):
                for b in range(NB):
                    gather(sv, b, bufs[b], gsems[b])

                @pl.loop(0, ROUNDS)
                def _(r):
                    for b in range(NB):
                        jj = NB * r + b
                        wait_g(bufs[b], gsems[b])
                        scat(dv, jj, bufs[b], ssems[b])

                        @pl.when(r < ROUNDS - 1)
                        def _(b=b, jj=jj):
                            wait_s(bufs[b], ssems[b])
                            gather(sv, jj + NB, bufs[b], gsems[b])

                for b in range(NB):
                    wait_s(bufs[b], ssems[b])

            csl0 = pl.ds(ebase, CHE)
            pltpu.sync_copy(e_hbm.at[0, csl0], srcA)
            pltpu.sync_copy(e_hbm.at[1, csl0], dstA)

            @pl.loop(0, NCH // 2)
            def _(m):
                k0 = 2 * m

                @pl.when(m > 0)
                def _():
                    wait_idx(srcA, dstA, isA)

                prefetch(k0 + 1, srcB, dstB, isB)
                rounds(srcA, dstA)

                wait_idx(srcB, dstB, isB)

                @pl.when(m < NCH // 2 - 1)
                def _():
                    prefetch(k0 + 2, srcA, dstA, isA)

                rounds(srcB, dstB)

        @pl.when(c == 0)
        def _():
            edge_loop(y0, e0)

        @pl.when(c == 1)
        def _():
            edge_loop(y1, e1)

        plsc.subcore_barrier()

        @pl.when(c == 0)
        def _():
            pltpu.sync_copy(agg_sh.at[sl], agg0.at[sl])

            @pl.when(s == NT - 1)
            def _():
                pltpu.sync_copy(agg_sh.at[tsl], agg0.at[tsl])

        @pl.when(c == 1)
        def _():
            pltpu.sync_copy(agg_sh.at[sl], agg1.at[sl])

            @pl.when(s == NT - 1)
            def _():
                pltpu.sync_copy(agg_sh.at[tsl], agg1.at[tsl])

    return sc_degrees, sc_scatter


_RB = 1000


def _y_body(x_ref, d0_ref, d1_ref, w0_ref, w1_ref, y0_ref, y1_ref):
    c0 = lax.rsqrt(jnp.maximum(d0_ref[:, 0:1], 1.0))
    c1 = lax.rsqrt(jnp.maximum(d1_ref[:, 0:1], 1.0))
    xb = x_ref[...]
    y0_ref[...] = jnp.dot(xb * c0, w0_ref[...], preferred_element_type=jnp.float32)
    y1_ref[...] = jnp.dot(xb * c1, w1_ref[...], preferred_element_type=jnp.float32)


def _tc_prepare_y(x, d0, d1, W0, W1):
    return pl.pallas_call(
        _y_body,
        grid=(N_NODES // _RB,),
        in_specs=[
            pl.BlockSpec((_RB, D), lambda i: (i, 0)),
            pl.BlockSpec((_RB, DEGW), lambda i: (i, 0)),
            pl.BlockSpec((_RB, DEGW), lambda i: (i, 0)),
            pl.BlockSpec((D, D), lambda i: (0, 0)),
            pl.BlockSpec((D, D), lambda i: (0, 0)),
        ],
        out_specs=[
            pl.BlockSpec((_RB, D), lambda i: (i, 0)),
            pl.BlockSpec((_RB, D), lambda i: (i, 0)),
        ],
        out_shape=[
            jax.ShapeDtypeStruct((N_NODES, D), jnp.float32),
            jax.ShapeDtypeStruct((N_NODES, D), jnp.float32),
        ],
    )(x, d0, d1, W0, W1)


def _fin_body(a0_ref, a1_ref, d0_ref, d1_ref, b0_ref, b1_ref, o_ref):
    s0 = lax.rsqrt(jnp.maximum(d0_ref[:, 0:1], 1.0))
    s1 = lax.rsqrt(jnp.maximum(d1_ref[:, 0:1], 1.0))
    o_ref[...] = a0_ref[...] * s0 + a1_ref[...] * s1 + b0_ref[...] + b1_ref[...]


def _tc_finalize(agg0, agg1, d0, d1, b0, b1):
    return pl.pallas_call(
        _fin_body,
        grid=(N_NODES // _RB,),
        in_specs=[
            pl.BlockSpec((_RB, D), lambda i: (i, 0)),
            pl.BlockSpec((_RB, D), lambda i: (i, 0)),
            pl.BlockSpec((_RB, DEGW), lambda i: (i, 0)),
            pl.BlockSpec((_RB, DEGW), lambda i: (i, 0)),
            pl.BlockSpec((1, D), lambda i: (0, 0)),
            pl.BlockSpec((1, D), lambda i: (0, 0)),
        ],
        out_specs=pl.BlockSpec((_RB, D), lambda i: (i, 0)),
        out_shape=jax.ShapeDtypeStruct((N_NODES, D), jnp.float32),
    )(agg0, agg1, d0, d1, b0, b1)


def kernel(x, edge_index_rel0, edge_index_rel1, W0, b0, W1, b1):
    e0 = edge_index_rel0.astype(jnp.int32)
    e1 = edge_index_rel1.astype(jnp.int32)
    ones_hbm = jnp.ones((BD, DEGW), jnp.float32)
    zeros_deg = jnp.zeros((ROWS_PT, DEGW), jnp.float32)
    zeros_agg = jnp.zeros((ROWS_PT, D), jnp.float32)

    sc_degrees, sc_scatter = _sc_kernels()
    outdeg0, indeg0, outdeg1, indeg1 = sc_degrees(e0, e1, ones_hbm, zeros_deg)
    y0, y1 = _tc_prepare_y(x, outdeg0, outdeg1, W0, W1)
    agg0, agg1 = sc_scatter(y0, y1, e0, e1, zeros_agg)
    return _tc_finalize(agg0, agg1, indeg0, indeg1,
                        b0.reshape(1, D), b1.reshape(1, D))


# speedup vs baseline: 1.5382x; 1.0138x over previous
import functools

import jax
import jax.numpy as jnp
from jax import lax
from jax.experimental import pallas as pl
from jax.experimental.pallas import tpu as pltpu
from jax.experimental.pallas import tpu_sc as plsc

N_NODES = 10000
D = 128
N_EDGES = 320000
NT = 16
EPT = N_EDGES // NT
BD = 128
RPTD = EPT // BD
DTAIL = EPT - RPTD * BD
FIRE = 4
QD = RPTD // FIRE
BS = 40
RPT_S = EPT // BS
NB = 5
CH = 50
CHE = CH * BS
NCH = RPT_S // CH
ROUNDS = CH // NB
ROWS_PT = 624
TAIL_BASE = ROWS_PT * NT
TAIL = N_NODES - TAIL_BASE
DEGW = 16

_deg_struct = jax.ShapeDtypeStruct((N_NODES, DEGW), jnp.float32)
_agg_struct = jax.ShapeDtypeStruct((N_NODES, D), jnp.float32)


@functools.cache
def _sc_kernels():
    mesh = plsc.VectorSubcoreMesh(core_axis_name="c", subcore_axis_name="s")
    params = pltpu.CompilerParams(use_tc_tiling_on_sc=False)

    @functools.partial(
        pl.kernel,
        out_type=(_deg_struct, _deg_struct, _deg_struct, _deg_struct),
        mesh=mesh,
        compiler_params=params,
        scratch_types=(
            pltpu.VMEM_SHARED((N_NODES, DEGW), jnp.float32),
            pltpu.VMEM_SHARED((N_NODES, DEGW), jnp.float32),
            pltpu.VMEM((EPT,), jnp.int32),
            pltpu.VMEM((EPT,), jnp.int32),
            pltpu.VMEM((BD, DEGW), jnp.float32),
            pltpu.SemaphoreType.DMA,
        ),
    )
    def sc_degrees(e0, e1, ones_hbm, zeros_hbm,
                   outdeg0, indeg0, outdeg1, indeg1,
                   deg_out_sh, deg_in_sh, src_v, dst_v, ones_v, dsem):
        c = lax.axis_index("c")
        s = lax.axis_index("s")
        sl = pl.ds(s * ROWS_PT, ROWS_PT)
        tsl = pl.ds(TAIL_BASE, TAIL)
        pltpu.sync_copy(zeros_hbm.at[pl.ds(0, ROWS_PT)], deg_out_sh.at[sl])
        pltpu.sync_copy(zeros_hbm.at[pl.ds(0, ROWS_PT)], deg_in_sh.at[sl])

        @pl.when(s == NT - 1)
        def _():
            pltpu.sync_copy(zeros_hbm.at[pl.ds(0, TAIL)], deg_out_sh.at[tsl])
            pltpu.sync_copy(zeros_hbm.at[pl.ds(0, TAIL)], deg_in_sh.at[tsl])

        pltpu.sync_copy(ones_hbm, ones_v)
        esl = pl.ds(s * EPT, EPT)

        @pl.when(c == 0)
        def _():
            pltpu.sync_copy(e0.at[0, esl], src_v)
            pltpu.sync_copy(e0.at[1, esl], dst_v)

        @pl.when(c == 1)
        def _():
            pltpu.sync_copy(e1.at[0, esl], src_v)
            pltpu.sync_copy(e1.at[1, esl], dst_v)

        plsc.subcore_barrier()

        def fire(q):
            for t in range(FIRE):
                isl = pl.ds((FIRE * q + t) * BD, BD)
                pltpu.async_copy(ones_v, deg_out_sh.at[src_v.at[isl]],
                                 dsem, add=True)
                pltpu.async_copy(ones_v, deg_in_sh.at[dst_v.at[isl]],
                                 dsem, add=True)

        def drain():
            for _t in range(2 * FIRE):
                pltpu.make_async_copy(ones_v, deg_out_sh.at[src_v.at[pl.ds(0, BD)]],
                                      dsem).wait()

        fire(0)

        @pl.loop(1, QD)
        def _(q):
            fire(q)
            drain()

        drain()

        tisl = pl.ds(RPTD * BD, DTAIL)
        tones = ones_v.at[pl.ds(0, DTAIL)]
        pltpu.sync_copy(tones, deg_out_sh.at[src_v.at[tisl]], add=True)
        pltpu.sync_copy(tones, deg_in_sh.at[dst_v.at[tisl]], add=True)

        plsc.subcore_barrier()

        @pl.when(c == 0)
        def _():
            pltpu.sync_copy(deg_out_sh.at[sl], outdeg0.at[sl])
            pltpu.sync_copy(deg_in_sh.at[sl], indeg0.at[sl])

            @pl.when(s == NT - 1)
            def _():
                pltpu.sync_copy(deg_out_sh.at[tsl], outdeg0.at[tsl])
                pltpu.sync_copy(deg_in_sh.at[tsl], indeg0.at[tsl])

        @pl.when(c == 1)
        def _():
            pltpu.sync_copy(deg_out_sh.at[sl], outdeg1.at[sl])
            pltpu.sync_copy(deg_in_sh.at[sl], indeg1.at[sl])

            @pl.when(s == NT - 1)
            def _():
                pltpu.sync_copy(deg_out_sh.at[tsl], outdeg1.at[tsl])
                pltpu.sync_copy(deg_in_sh.at[tsl], indeg1.at[tsl])

    scatter_scratch = (
        [pltpu.VMEM_SHARED((N_NODES, D), jnp.float32)]
        + [pltpu.VMEM((CHE,), jnp.int32) for _ in range(4)]
        + [pltpu.VMEM((BS, D), jnp.float32) for _ in range(NB)]
        + [pltpu.SemaphoreType.DMA for _ in range(2 * NB + 2)]
    )

    @functools.partial(
        pl.kernel,
        out_type=(_agg_struct, _agg_struct),
        mesh=mesh,
        compiler_params=params,
        scratch_types=tuple(scatter_scratch),
    )
    def sc_scatter(y0, y1, e0, e1, zeros_hbm,
                   agg0, agg1,
                   agg_sh, srcA, dstA, srcB, dstB,
                   r0, r1, r2, r3, r4,
                   g0, g1, g2, g3, g4,
                   s0, s1, s2, s3, s4, isA, isB):
        c = lax.axis_index("c")
        s = lax.axis_index("s")
        bufs = (r0, r1, r2, r3, r4)
        gsems = (g0, g1, g2, g3, g4)
        ssems = (s0, s1, s2, s3, s4)
        sl = pl.ds(s * ROWS_PT, ROWS_PT)
        tsl = pl.ds(TAIL_BASE, TAIL)
        pltpu.sync_copy(zeros_hbm.at[pl.ds(0, ROWS_PT)], agg_sh.at[sl])

        @pl.when(s == NT - 1)
        def _():
            pltpu.sync_copy(zeros_hbm.at[pl.ds(0, TAIL)], agg_sh.at[tsl])

        plsc.subcore_barrier()

        ebase = s * EPT

        def edge_loop(y_hbm, e_hbm):
            def gather(sv, jj, buf, sem):
                pltpu.async_copy(y_hbm.at[sv.at[pl.ds(jj * BS, BS)]], buf, sem)

            def wait_g(buf, sem):
                pltpu.make_async_copy(y_hbm.at[srcA.at[pl.ds(0, BS)]],
                                      buf, sem).wait()

            def scat(dv, jj, buf, sem):
                pltpu.async_copy(buf, agg_sh.at[dv.at[pl.ds(jj * BS, BS)]],
                                 sem, add=True)

            def wait_s(buf, sem):
                pltpu.make_async_copy(buf, agg_sh.at[dstA.at[pl.ds(0, BS)]],
                                      sem).wait()

            def prefetch(k, sv, dv, isem):
                csl = pl.ds(ebase + k * CHE, CHE)
                pltpu.async_copy(e_hbm.at[0, csl], sv, isem)
                pltpu.async_copy(e_hbm.at[1, csl], dv, isem)

            def wait_idx(sv, dv, isem):
                pltpu.make_async_copy(e_hbm.at[0, pl.ds(0, CHE)], sv, isem).wait()
                pltpu.make_async_copy(e_hbm.at[0, pl.ds(0, CHE)], dv, isem).wait()

            def rounds(sv, dv):
                for b in range(NB):
                    gather(sv, b, bufs[b], gsems[b])

                @pl.loop(0, ROUNDS)
                def _(r):
                    for b in range(NB):
                        jj = NB * r + b
                        wait_g(bufs[b], gsems[b])
                        scat(dv, jj, bufs[b], ssems[b])

                        @pl.when(r < ROUNDS - 1)
                        def _(b=b, jj=jj):
                            wait_s(bufs[b], ssems[b])
                            gather(sv, jj + NB, bufs[b], gsems[b])

                for b in range(NB):
                    wait_s(bufs[b], ssems[b])

            csl0 = pl.ds(ebase, CHE)
            pltpu.sync_copy(e_hbm.at[0, csl0], srcA)
            pltpu.sync_copy(e_hbm.at[1, csl0], dstA)

            @pl.loop(0, NCH // 2)
            def _(m):
                k0 = 2 * m

                @pl.when(m > 0)
                def _():
                    wait_idx(srcA, dstA, isA)

                prefetch(k0 + 1, srcB, dstB, isB)
                rounds(srcA, dstA)

                wait_idx(srcB, dstB, isB)

                @pl.when(m < NCH // 2 - 1)
                def _():
                    prefetch(k0 + 2, srcA, dstA, isA)

                rounds(srcB, dstB)

        @pl.when(c == 0)
        def _():
            edge_loop(y0, e0)

        @pl.when(c == 1)
        def _():
            edge_loop(y1, e1)

        plsc.subcore_barrier()

        @pl.when(c == 0)
        def _():
            pltpu.sync_copy(agg_sh.at[sl], agg0.at[sl])

            @pl.when(s == NT - 1)
            def _():
                pltpu.sync_copy(agg_sh.at[tsl], agg0.at[tsl])

        @pl.when(c == 1)
        def _():
            pltpu.sync_copy(agg_sh.at[sl], agg1.at[sl])

            @pl.when(s == NT - 1)
            def _():
                pltpu.sync_copy(agg_sh.at[tsl], agg1.at[tsl])

    return sc_degrees, sc_scatter


_RB = 2000


def _y_body(x_ref, d0_ref, d1_ref, w0_ref, w1_ref, y0_ref, y1_ref):
    c0 = lax.rsqrt(jnp.maximum(d0_ref[:, 0:1], 1.0))
    c1 = lax.rsqrt(jnp.maximum(d1_ref[:, 0:1], 1.0))
    xb = x_ref[...]
    y0_ref[...] = jnp.dot(xb * c0, w0_ref[...], preferred_element_type=jnp.float32)
    y1_ref[...] = jnp.dot(xb * c1, w1_ref[...], preferred_element_type=jnp.float32)


def _tc_prepare_y(x, d0, d1, W0, W1):
    return pl.pallas_call(
        _y_body,
        grid=(N_NODES // _RB,),
        in_specs=[
            pl.BlockSpec((_RB, D), lambda i: (i, 0)),
            pl.BlockSpec((_RB, DEGW), lambda i: (i, 0)),
            pl.BlockSpec((_RB, DEGW), lambda i: (i, 0)),
            pl.BlockSpec((D, D), lambda i: (0, 0)),
            pl.BlockSpec((D, D), lambda i: (0, 0)),
        ],
        out_specs=[
            pl.BlockSpec((_RB, D), lambda i: (i, 0)),
            pl.BlockSpec((_RB, D), lambda i: (i, 0)),
        ],
        out_shape=[
            jax.ShapeDtypeStruct((N_NODES, D), jnp.float32),
            jax.ShapeDtypeStruct((N_NODES, D), jnp.float32),
        ],
    )(x, d0, d1, W0, W1)


def _fin_body(a0_ref, a1_ref, d0_ref, d1_ref, b0_ref, b1_ref, o_ref):
    s0 = lax.rsqrt(jnp.maximum(d0_ref[:, 0:1], 1.0))
    s1 = lax.rsqrt(jnp.maximum(d1_ref[:, 0:1], 1.0))
    o_ref[...] = a0_ref[...] * s0 + a1_ref[...] * s1 + b0_ref[...] + b1_ref[...]


def _tc_finalize(agg0, agg1, d0, d1, b0, b1):
    return pl.pallas_call(
        _fin_body,
        grid=(N_NODES // _RB,),
        in_specs=[
            pl.BlockSpec((_RB, D), lambda i: (i, 0)),
            pl.BlockSpec((_RB, D), lambda i: (i, 0)),
            pl.BlockSpec((_RB, DEGW), lambda i: (i, 0)),
            pl.BlockSpec((_RB, DEGW), lambda i: (i, 0)),
            pl.BlockSpec((1, D), lambda i: (0, 0)),
            pl.BlockSpec((1, D), lambda i: (0, 0)),
        ],
        out_specs=pl.BlockSpec((_RB, D), lambda i: (i, 0)),
        out_shape=jax.ShapeDtypeStruct((N_NODES, D), jnp.float32),
    )(agg0, agg1, d0, d1, b0, b1)


def kernel(x, edge_index_rel0, edge_index_rel1, W0, b0, W1, b1):
    e0 = edge_index_rel0.astype(jnp.int32)
    e1 = edge_index_rel1.astype(jnp.int32)
    ones_hbm = jnp.ones((BD, DEGW), jnp.float32)
    zeros_deg = jnp.zeros((ROWS_PT, DEGW), jnp.float32)
    zeros_agg = jnp.zeros((ROWS_PT, D), jnp.float32)

    sc_degrees, sc_scatter = _sc_kernels()
    outdeg0, indeg0, outdeg1, indeg1 = sc_degrees(e0, e1, ones_hbm, zeros_deg)
    y0, y1 = _tc_prepare_y(x, outdeg0, outdeg1, W0, W1)
    agg0, agg1 = sc_scatter(y0, y1, e0, e1, zeros_agg)
    return _tc_finalize(agg0, agg1, indeg0, indeg1,
                        b0.reshape(1, D), b1.reshape(1, D))


# speedup vs baseline: 1.5816x; 1.0282x over previous
import functools

import jax
import jax.numpy as jnp
from jax import lax
from jax.experimental import pallas as pl
from jax.experimental.pallas import tpu as pltpu
from jax.experimental.pallas import tpu_sc as plsc

N_NODES = 10000
D = 128
N_EDGES = 320000
NT = 16
EPT = N_EDGES // NT
BD = 128
RPTD = EPT // BD
DTAIL = EPT - RPTD * BD
FIRE = 4
QD = RPTD // FIRE
BS = 40
RPT_S = EPT // BS
NB = 5
CH = 125
CHE = CH * BS
NCH = RPT_S // CH
ROUNDS = CH // NB
ROWS_PT = 624
TAIL_BASE = ROWS_PT * NT
TAIL = N_NODES - TAIL_BASE
DEGW = 16

_deg_struct = jax.ShapeDtypeStruct((N_NODES, DEGW), jnp.float32)
_agg_struct = jax.ShapeDtypeStruct((N_NODES, D), jnp.float32)


@functools.cache
def _sc_kernels():
    mesh = plsc.VectorSubcoreMesh(core_axis_name="c", subcore_axis_name="s")
    params = pltpu.CompilerParams(use_tc_tiling_on_sc=False)

    @functools.partial(
        pl.kernel,
        out_type=(_deg_struct, _deg_struct, _deg_struct, _deg_struct),
        mesh=mesh,
        compiler_params=params,
        scratch_types=(
            pltpu.VMEM_SHARED((N_NODES, DEGW), jnp.float32),
            pltpu.VMEM_SHARED((N_NODES, DEGW), jnp.float32),
            pltpu.VMEM((EPT,), jnp.int32),
            pltpu.VMEM((EPT,), jnp.int32),
            pltpu.VMEM((BD, DEGW), jnp.float32),
            pltpu.SemaphoreType.DMA,
        ),
    )
    def sc_degrees(e0, e1, ones_hbm, zeros_hbm,
                   outdeg0, indeg0, outdeg1, indeg1,
                   deg_out_sh, deg_in_sh, src_v, dst_v, ones_v, dsem):
        c = lax.axis_index("c")
        s = lax.axis_index("s")
        sl = pl.ds(s * ROWS_PT, ROWS_PT)
        tsl = pl.ds(TAIL_BASE, TAIL)
        pltpu.sync_copy(zeros_hbm.at[pl.ds(0, ROWS_PT)], deg_out_sh.at[sl])
        pltpu.sync_copy(zeros_hbm.at[pl.ds(0, ROWS_PT)], deg_in_sh.at[sl])

        @pl.when(s == NT - 1)
        def _():
            pltpu.sync_copy(zeros_hbm.at[pl.ds(0, TAIL)], deg_out_sh.at[tsl])
            pltpu.sync_copy(zeros_hbm.at[pl.ds(0, TAIL)], deg_in_sh.at[tsl])

        pltpu.sync_copy(ones_hbm, ones_v)
        esl = pl.ds(s * EPT, EPT)

        @pl.when(c == 0)
        def _():
            pltpu.sync_copy(e0.at[0, esl], src_v)
            pltpu.sync_copy(e0.at[1, esl], dst_v)

        @pl.when(c == 1)
        def _():
            pltpu.sync_copy(e1.at[0, esl], src_v)
            pltpu.sync_copy(e1.at[1, esl], dst_v)

        plsc.subcore_barrier()

        def fire(q):
            for t in range(FIRE):
                isl = pl.ds((FIRE * q + t) * BD, BD)
                pltpu.async_copy(ones_v, deg_out_sh.at[src_v.at[isl]],
                                 dsem, add=True)
                pltpu.async_copy(ones_v, deg_in_sh.at[dst_v.at[isl]],
                                 dsem, add=True)

        def drain():
            for _t in range(2 * FIRE):
                pltpu.make_async_copy(ones_v, deg_out_sh.at[src_v.at[pl.ds(0, BD)]],
                                      dsem).wait()

        fire(0)

        @pl.loop(1, QD)
        def _(q):
            fire(q)
            drain()

        drain()

        tisl = pl.ds(RPTD * BD, DTAIL)
        tones = ones_v.at[pl.ds(0, DTAIL)]
        pltpu.sync_copy(tones, deg_out_sh.at[src_v.at[tisl]], add=True)
        pltpu.sync_copy(tones, deg_in_sh.at[dst_v.at[tisl]], add=True)

        plsc.subcore_barrier()

        @pl.when(c == 0)
        def _():
            pltpu.sync_copy(deg_out_sh.at[sl], outdeg0.at[sl])
            pltpu.sync_copy(deg_in_sh.at[sl], indeg0.at[sl])

            @pl.when(s == NT - 1)
            def _():
                pltpu.sync_copy(deg_out_sh.at[tsl], outdeg0.at[tsl])
                pltpu.sync_copy(deg_in_sh.at[tsl], indeg0.at[tsl])

        @pl.when(c == 1)
        def _():
            pltpu.sync_copy(deg_out_sh.at[sl], outdeg1.at[sl])
            pltpu.sync_copy(deg_in_sh.at[sl], indeg1.at[sl])

            @pl.when(s == NT - 1)
            def _():
                pltpu.sync_copy(deg_out_sh.at[tsl], outdeg1.at[tsl])
                pltpu.sync_copy(deg_in_sh.at[tsl], indeg1.at[tsl])

    scatter_scratch = (
        [pltpu.VMEM_SHARED((N_NODES, D), jnp.float32)]
        + [pltpu.VMEM((CHE,), jnp.int32) for _ in range(4)]
        + [pltpu.VMEM((BS, D), jnp.float32) for _ in range(NB)]
        + [pltpu.SemaphoreType.DMA for _ in range(2 * NB + 2)]
    )

    @functools.partial(
        pl.kernel,
        out_type=(_agg_struct, _agg_struct),
        mesh=mesh,
        compiler_params=params,
        scratch_types=tuple(scatter_scratch),
    )
    def sc_scatter(y0, y1, e0, e1, zeros_hbm,
                   agg0, agg1,
                   agg_sh, srcA, dstA, srcB, dstB,
                   r0, r1, r2, r3, r4,
                   g0, g1, g2, g3, g4,
                   s0, s1, s2, s3, s4, isA, isB):
        c = lax.axis_index("c")
        s = lax.axis_index("s")
        bufs = (r0, r1, r2, r3, r4)
        gsems = (g0, g1, g2, g3, g4)
        ssems = (s0, s1, s2, s3, s4)
        sl = pl.ds(s * ROWS_PT, ROWS_PT)
        tsl = pl.ds(TAIL_BASE, TAIL)
        pltpu.sync_copy(zeros_hbm.at[pl.ds(0, ROWS_PT)], agg_sh.at[sl])

        @pl.when(s == NT - 1)
        def _():
            pltpu.sync_copy(zeros_hbm.at[pl.ds(0, TAIL)], agg_sh.at[tsl])

        plsc.subcore_barrier()

        ebase = s * EPT

        def edge_loop(y_hbm, e_hbm):
            def gather(sv, jj, buf, sem):
                pltpu.async_copy(y_hbm.at[sv.at[pl.ds(jj * BS, BS)]], buf, sem)

            def wait_g(buf, sem):
                pltpu.make_async_copy(y_hbm.at[srcA.at[pl.ds(0, BS)]],
                                      buf, sem).wait()

            def scat(dv, jj, buf, sem):
                pltpu.async_copy(buf, agg_sh.at[dv.at[pl.ds(jj * BS, BS)]],
                                 sem, add=True)

            def wait_s(buf, sem):
                pltpu.make_async_copy(buf, agg_sh.at[dstA.at[pl.ds(0, BS)]],
                                      sem).wait()

            def prefetch(k, sv, dv, isem):
                csl = pl.ds(ebase + k * CHE, CHE)
                pltpu.async_copy(e_hbm.at[0, csl], sv, isem)
                pltpu.async_copy(e_hbm.at[1, csl], dv, isem)

            def wait_idx(sv, dv, isem):
                pltpu.make_async_copy(e_hbm.at[0, pl.ds(0, CHE)], sv, isem).wait()
                pltpu.make_async_copy(e_hbm.at[0, pl.ds(0, CHE)], dv, isem).wait()

            def rounds(sv, dv):
                for b in range(NB):
                    gather(sv, b, bufs[b], gsems[b])

                @pl.loop(0, ROUNDS)
                def _(r):
                    for b in range(NB):
                        jj = NB * r + b
                        wait_g(bufs[b], gsems[b])
                        scat(dv, jj, bufs[b], ssems[b])

                        @pl.when(r < ROUNDS - 1)
                        def _(b=b, jj=jj):
                            wait_s(bufs[b], ssems[b])
                            gather(sv, jj + NB, bufs[b], gsems[b])

                for b in range(NB):
                    wait_s(bufs[b], ssems[b])

            csl0 = pl.ds(ebase, CHE)
            pltpu.sync_copy(e_hbm.at[0, csl0], srcA)
            pltpu.sync_copy(e_hbm.at[1, csl0], dstA)

            @pl.loop(0, NCH // 2)
            def _(m):
                k0 = 2 * m

                @pl.when(m > 0)
                def _():
                    wait_idx(srcA, dstA, isA)

                prefetch(k0 + 1, srcB, dstB, isB)
                rounds(srcA, dstA)

                wait_idx(srcB, dstB, isB)

                @pl.when(m < NCH // 2 - 1)
                def _():
                    prefetch(k0 + 2, srcA, dstA, isA)

                rounds(srcB, dstB)

        @pl.when(c == 0)
        def _():
            edge_loop(y0, e0)

        @pl.when(c == 1)
        def _():
            edge_loop(y1, e1)

        plsc.subcore_barrier()

        @pl.when(c == 0)
        def _():
            pltpu.sync_copy(agg_sh.at[sl], agg0.at[sl])

            @pl.when(s == NT - 1)
            def _():
                pltpu.sync_copy(agg_sh.at[tsl], agg0.at[tsl])

        @pl.when(c == 1)
        def _():
            pltpu.sync_copy(agg_sh.at[sl], agg1.at[sl])

            @pl.when(s == NT - 1)
            def _():
                pltpu.sync_copy(agg_sh.at[tsl], agg1.at[tsl])

    return sc_degrees, sc_scatter


_RB = 2000


def _y_body(x_ref, d0_ref, d1_ref, w0_ref, w1_ref, y0_ref, y1_ref):
    c0 = lax.rsqrt(jnp.maximum(d0_ref[:, 0:1], 1.0))
    c1 = lax.rsqrt(jnp.maximum(d1_ref[:, 0:1], 1.0))
    xb = x_ref[...]
    y0_ref[...] = jnp.dot(xb * c0, w0_ref[...], preferred_element_type=jnp.float32)
    y1_ref[...] = jnp.dot(xb * c1, w1_ref[...], preferred_element_type=jnp.float32)


def _tc_prepare_y(x, d0, d1, W0, W1):
    return pl.pallas_call(
        _y_body,
        grid=(N_NODES // _RB,),
        in_specs=[
            pl.BlockSpec((_RB, D), lambda i: (i, 0)),
            pl.BlockSpec((_RB, DEGW), lambda i: (i, 0)),
            pl.BlockSpec((_RB, DEGW), lambda i: (i, 0)),
            pl.BlockSpec((D, D), lambda i: (0, 0)),
            pl.BlockSpec((D, D), lambda i: (0, 0)),
        ],
        out_specs=[
            pl.BlockSpec((_RB, D), lambda i: (i, 0)),
            pl.BlockSpec((_RB, D), lambda i: (i, 0)),
        ],
        out_shape=[
            jax.ShapeDtypeStruct((N_NODES, D), jnp.float32),
            jax.ShapeDtypeStruct((N_NODES, D), jnp.float32),
        ],
    )(x, d0, d1, W0, W1)


def _fin_body(a0_ref, a1_ref, d0_ref, d1_ref, b0_ref, b1_ref, o_ref):
    s0 = lax.rsqrt(jnp.maximum(d0_ref[:, 0:1], 1.0))
    s1 = lax.rsqrt(jnp.maximum(d1_ref[:, 0:1], 1.0))
    o_ref[...] = a0_ref[...] * s0 + a1_ref[...] * s1 + b0_ref[...] + b1_ref[...]


def _tc_finalize(agg0, agg1, d0, d1, b0, b1):
    return pl.pallas_call(
        _fin_body,
        grid=(N_NODES // _RB,),
        in_specs=[
            pl.BlockSpec((_RB, D), lambda i: (i, 0)),
            pl.BlockSpec((_RB, D), lambda i: (i, 0)),
            pl.BlockSpec((_RB, DEGW), lambda i: (i, 0)),
            pl.BlockSpec((_RB, DEGW), lambda i: (i, 0)),
            pl.BlockSpec((1, D), lambda i: (0, 0)),
            pl.BlockSpec((1, D), lambda i: (0, 0)),
        ],
        out_specs=pl.BlockSpec((_RB, D), lambda i: (i, 0)),
        out_shape=jax.ShapeDtypeStruct((N_NODES, D), jnp.float32),
    )(agg0, agg1, d0, d1, b0, b1)


def kernel(x, edge_index_rel0, edge_index_rel1, W0, b0, W1, b1):
    e0 = edge_index_rel0.astype(jnp.int32)
    e1 = edge_index_rel1.astype(jnp.int32)
    ones_hbm = jnp.ones((BD, DEGW), jnp.float32)
    zeros_deg = jnp.zeros((ROWS_PT, DEGW), jnp.float32)
    zeros_agg = jnp.zeros((ROWS_PT, D), jnp.float32)

    sc_degrees, sc_scatter = _sc_kernels()
    outdeg0, indeg0, outdeg1, indeg1 = sc_degrees(e0, e1, ones_hbm, zeros_deg)
    y0, y1 = _tc_prepare_y(x, outdeg0, outdeg1, W0, W1)
    agg0, agg1 = sc_scatter(y0, y1, e0, e1, zeros_agg)
    return _tc_finalize(agg0, agg1, indeg0, indeg1,
                        b0.reshape(1, D), b1.reshape(1, D))
